# baseline (speedup 1.0000x reference)
import functools

import jax
import jax.numpy as jnp
from jax import lax
from jax.experimental import pallas as pl
from jax.experimental.pallas import tpu as pltpu
from jax.experimental.pallas import tpu_sc as plsc

N_NODES = 10000
N_EDGES = 320000
D_FEAT = 128
D_HALF = D_FEAT // 2
D_EDGE = 16
OUT_DIM = 128

NC = 2
NS = 16
CHUNK = 64
NBUF = 4
N_EPAD = 327680
EDGES_PER_TILE = N_EPAD // NS
NCH = EDGES_PER_TILE // CHUNK
N_PAD = 10240
ROWS_PER_TILE = N_PAD // NS

_sc_mesh = plsc.VectorSubcoreMesh(
    core_axis_name="c", subcore_axis_name="s", num_cores=NC)


@functools.partial(
    pl.kernel,
    out_type=(
        jax.ShapeDtypeStruct((NC, N_PAD, D_HALF), jnp.float32),
        jax.ShapeDtypeStruct((N_PAD, D_EDGE), jnp.float32),
        jax.ShapeDtypeStruct((N_PAD, 16), jnp.float32),
    ),
    mesh=_sc_mesh,
    compiler_params=pltpu.CompilerParams(use_tc_tiling_on_sc=False),
    scratch_types=(
        pltpu.VMEM((NCH, CHUNK), jnp.int32),
        pltpu.VMEM((NCH, CHUNK), jnp.int32),
        pltpu.VMEM((CHUNK, D_HALF), jnp.float32),
        pltpu.VMEM((CHUNK, D_HALF), jnp.float32),
        pltpu.VMEM((CHUNK, D_HALF), jnp.float32),
        pltpu.VMEM((CHUNK, D_HALF), jnp.float32),
        pltpu.VMEM((CHUNK, D_EDGE), jnp.float32),
        pltpu.VMEM((CHUNK, D_EDGE), jnp.float32),
        pltpu.VMEM((CHUNK, D_EDGE), jnp.float32),
        pltpu.VMEM((CHUNK, D_EDGE), jnp.float32),
        pltpu.VMEM((CHUNK, 16), jnp.float32),
        pltpu.VMEM_SHARED((N_PAD, D_HALF), jnp.float32),
        pltpu.VMEM_SHARED((N_PAD, D_EDGE), jnp.float32),
        pltpu.VMEM_SHARED((N_PAD, 16), jnp.float32),
        pltpu.SemaphoreType.DMA,
        pltpu.SemaphoreType.DMA,
        pltpu.SemaphoreType.DMA,
        pltpu.SemaphoreType.DMA,
        pltpu.SemaphoreType.DMA,
        pltpu.SemaphoreType.DMA,
        pltpu.SemaphoreType.DMA,
        pltpu.SemaphoreType.DMA,
        pltpu.SemaphoreType.DMA,
        pltpu.SemaphoreType.DMA,
        pltpu.SemaphoreType.DMA,
        pltpu.SemaphoreType.DMA,
    ),
)
def _sc_accumulate(src_hbm, dst_hbm, x_hbm, ef_hbm, z64_hbm, z16_hbm, ones_hbm,
                   accx_hbm, acce_hbm, accc_hbm,
                   src_v, dst_v, xb0, xb1, xb2, xb3, eb0, eb1, eb2, eb3,
                   onesbuf, shx, she, shc,
                   g0, g1, g2, g3, e0, e1, e2, e3, s0, s1, s2, s3):
    c = lax.axis_index("c")
    s = lax.axis_index("s")
    xbufs = (xb0, xb1, xb2, xb3)
    ebufs = (eb0, eb1, eb2, eb3)
    gsems = (g0, g1, g2, g3)
    esems = (e0, e1, e2, e3)
    ssems = (s0, s1, s2, s3)

    r0 = s * ROWS_PER_TILE
    pltpu.sync_copy(z64_hbm, shx.at[pl.ds(r0, ROWS_PER_TILE)])
    @pl.when(c == 0)
    def _():
        pltpu.sync_copy(z16_hbm, she.at[pl.ds(r0, ROWS_PER_TILE)])

    @pl.when(c == 1)
    def _():
        pltpu.sync_copy(z16_hbm, shc.at[pl.ds(r0, ROWS_PER_TILE)])
        pltpu.sync_copy(ones_hbm, onesbuf)

    pltpu.sync_copy(src_hbm.at[s], src_v)
    pltpu.sync_copy(dst_hbm.at[s], dst_v)
    plsc.subcore_barrier()

    ef_base = s * EDGES_PER_TILE

    def ef_off(ci):
        return jnp.minimum(ef_base + ci * CHUNK, N_EDGES - CHUNK)

    def issue_gather(ci, p):
        pltpu.async_copy(x_hbm.at[c].at[src_v.at[ci]], xbufs[p], gsems[p])

        @pl.when(c == 0)
        def _():
            pltpu.async_copy(
                ef_hbm.at[pl.ds(ef_off(ci), CHUNK)], ebufs[p], esems[p])

    def drain_gather(p):
        pltpu.make_async_copy(
            x_hbm.at[c].at[src_v.at[0]], xbufs[p], gsems[p]).wait()

        @pl.when(c == 0)
        def _():
            pltpu.make_async_copy(
                ef_hbm.at[pl.ds(0, CHUNK)], ebufs[p], esems[p]).wait()

    def issue_scatter(ci, p):
        pltpu.async_copy(xbufs[p], shx.at[dst_v.at[ci]], ssems[p], add=True)

        @pl.when(c == 0)
        def _():
            pltpu.async_copy(ebufs[p], she.at[dst_v.at[ci]], ssems[p],
                             add=True)

        @pl.when(c == 1)
        def _():
            pltpu.async_copy(onesbuf, shc.at[dst_v.at[ci]], ssems[p],
                             add=True)

    def drain_scatter(p):
        pltpu.make_async_copy(xbufs[p], shx.at[dst_v.at[0]], ssems[p]).wait()

        @pl.when(c == 0)
        def _():
            pltpu.make_async_copy(
                ebufs[p], she.at[dst_v.at[0]], ssems[p]).wait()

        @pl.when(c == 1)
        def _():
            pltpu.make_async_copy(
                onesbuf, shc.at[dst_v.at[0]], ssems[p]).wait()

    def process(ci, p, drain):
        drain_gather(p)
        issue_scatter(ci, p)
        pn = (p + 2) % NBUF
        if drain:
            drain_scatter(pn)
        issue_gather(jnp.minimum(ci + 2, NCH - 1), pn)

    issue_gather(0, 0)
    issue_gather(1, 1)
    process(0, 0, drain=False)
    process(1, 1, drain=False)
    process(2, 2, drain=True)
    process(3, 3, drain=True)

    def quad_body(j, carry):
        a = 4 * j
        process(a, 0, drain=True)
        process(a + 1, 1, drain=True)
        process(a + 2, 2, drain=True)
        process(a + 3, 3, drain=True)
        return carry

    lax.fori_loop(1, NCH // 4, quad_body, 0)

    drain_gather(0)
    drain_gather(1)
    drain_scatter(2)
    drain_scatter(3)

    plsc.subcore_barrier()

    pltpu.sync_copy(shx.at[pl.ds(r0, ROWS_PER_TILE)],
                    accx_hbm.at[c].at[pl.ds(r0, ROWS_PER_TILE)])
    @pl.when(c == 0)
    def _():
        pltpu.sync_copy(she.at[pl.ds(r0, ROWS_PER_TILE)],
                        acce_hbm.at[pl.ds(r0, ROWS_PER_TILE)])

    @pl.when(c == 1)
    def _():
        pltpu.sync_copy(shc.at[pl.ds(r0, ROWS_PER_TILE)],
                        accc_hbm.at[pl.ds(r0, ROWS_PER_TILE)])


def _finish_body(accx_ref, acce_ref, accc_ref, wx_ref, we_ref, b_ref, out_ref):
    acc0 = accx_ref[0, :N_NODES]
    acc1 = accx_ref[1, :N_NODES]
    acce = acce_ref[:N_NODES]
    counts = accc_ref[:N_NODES, 0:1]
    sums = jnp.dot(acc0, wx_ref[:D_HALF], preferred_element_type=jnp.float32)
    sums = sums + jnp.dot(acc1, wx_ref[D_HALF:], preferred_element_type=jnp.float32)
    sums = sums + jnp.dot(acce, we_ref[...], preferred_element_type=jnp.float32)
    sums = sums + counts * b_ref[...]
    out_ref[...] = sums / jnp.maximum(counts, 1.0)


_finish = pl.pallas_call(
    _finish_body,
    out_shape=jax.ShapeDtypeStruct((N_NODES, OUT_DIM), jnp.float32),
)


def kernel(x, edge_index, edge_features, W, b):
    ei = edge_index.astype(jnp.int32)
    n_pad_e = N_EPAD - N_EDGES
    pad_src = jnp.zeros((n_pad_e,), jnp.int32)
    pad_dst = N_NODES + jnp.arange(n_pad_e, dtype=jnp.int32) % (N_PAD - N_NODES)
    src = jnp.concatenate([ei[0], pad_src]).reshape(NS, NCH, CHUNK)
    dst = jnp.concatenate([ei[1], pad_dst]).reshape(NS, NCH, CHUNK)
    xsplit = x.reshape(N_NODES, NC, D_HALF).transpose(1, 0, 2)
    z64 = jnp.zeros((ROWS_PER_TILE, D_HALF), jnp.float32)
    z16 = jnp.zeros((ROWS_PER_TILE, 16), jnp.float32)
    ones = jnp.ones((CHUNK, 16), jnp.float32)
    accx, acce, accc = _sc_accumulate(src, dst, xsplit, edge_features,
                                      z64, z16, ones)
    wx = W[:D_FEAT]
    we = W[D_FEAT:]
    return _finish(accx, acce, accc, wx, we, b.reshape(1, OUT_DIM))


# speedup vs baseline: 1.6878x; 1.6878x over previous
import functools

import jax
import jax.numpy as jnp
from jax import lax
from jax.experimental import pallas as pl
from jax.experimental.pallas import tpu as pltpu
from jax.experimental.pallas import tpu_sc as plsc

N_NODES = 10000
N_EDGES = 320000
D_FEAT = 128
D_HALF = D_FEAT // 2
D_EDGE = 16
OUT_DIM = 128

NC = 2
NS = 16
CHUNK = 125
NBUF = 3
EDGES_PER_TILE = N_EDGES // NS
NCH = EDGES_PER_TILE // CHUNK
N_PAD = 10240
ROWS_PER_TILE = N_PAD // NS

_sc_mesh = plsc.VectorSubcoreMesh(
    core_axis_name="c", subcore_axis_name="s", num_cores=NC)


@functools.partial(
    pl.kernel,
    out_type=(
        jax.ShapeDtypeStruct((NC, N_PAD, D_HALF), jnp.float32),
        jax.ShapeDtypeStruct((N_PAD, D_EDGE), jnp.float32),
        jax.ShapeDtypeStruct((N_PAD, 16), jnp.float32),
    ),
    mesh=_sc_mesh,
    compiler_params=pltpu.CompilerParams(use_tc_tiling_on_sc=False),
    scratch_types=(
        pltpu.VMEM((NCH, CHUNK), jnp.int32),
        pltpu.VMEM((NCH, CHUNK), jnp.int32),
        pltpu.VMEM((CHUNK, D_HALF), jnp.float32),
        pltpu.VMEM((CHUNK, D_HALF), jnp.float32),
        pltpu.VMEM((CHUNK, D_HALF), jnp.float32),
        pltpu.VMEM((CHUNK, D_EDGE), jnp.float32),
        pltpu.VMEM((CHUNK, D_EDGE), jnp.float32),
        pltpu.VMEM((CHUNK, D_EDGE), jnp.float32),
        pltpu.VMEM((CHUNK, 16), jnp.float32),
        pltpu.VMEM_SHARED((N_PAD, D_HALF), jnp.float32),
        pltpu.VMEM_SHARED((N_PAD, 16), jnp.float32),
        pltpu.SemaphoreType.DMA,
        pltpu.SemaphoreType.DMA,
        pltpu.SemaphoreType.DMA,
        pltpu.SemaphoreType.DMA,
        pltpu.SemaphoreType.DMA,
        pltpu.SemaphoreType.DMA,
        pltpu.SemaphoreType.DMA,
        pltpu.SemaphoreType.DMA,
        pltpu.SemaphoreType.DMA,
    ),
)
def _sc_accumulate(src_hbm, dst_hbm, x_hbm, ef_hbm, z64_hbm, z16_hbm, ones_hbm,
                   accx_hbm, acce_hbm, accc_hbm,
                   src_v, dst_v, xb0, xb1, xb2, eb0, eb1, eb2,
                   onesbuf, shx, shm,
                   g0, g1, g2, e0, e1, e2, s0, s1, s2):
    c = lax.axis_index("c")
    s = lax.axis_index("s")
    xbufs = (xb0, xb1, xb2)
    ebufs = (eb0, eb1, eb2)
    gsems = (g0, g1, g2)
    esems = (e0, e1, e2)
    ssems = (s0, s1, s2)

    r0 = s * ROWS_PER_TILE
    pltpu.sync_copy(z64_hbm, shx.at[pl.ds(r0, ROWS_PER_TILE)])
    pltpu.sync_copy(z16_hbm, shm.at[pl.ds(r0, ROWS_PER_TILE)])

    @pl.when(c == 1)
    def _():
        pltpu.sync_copy(ones_hbm, onesbuf)

    pltpu.sync_copy(src_hbm.at[s], src_v)
    pltpu.sync_copy(dst_hbm.at[s], dst_v)
    plsc.subcore_barrier()

    ef_base = s * EDGES_PER_TILE

    def issue_gather(ci, p):
        pltpu.async_copy(x_hbm.at[c].at[src_v.at[ci]], xbufs[p], gsems[p])

        @pl.when(c == 0)
        def _():
            pltpu.async_copy(
                ef_hbm.at[pl.ds(ef_base + ci * CHUNK, CHUNK)],
                ebufs[p], esems[p])

    def drain_gather(p):
        pltpu.make_async_copy(
            x_hbm.at[c].at[src_v.at[0]], xbufs[p], gsems[p]).wait()

        @pl.when(c == 0)
        def _():
            pltpu.make_async_copy(
                ef_hbm.at[pl.ds(0, CHUNK)], ebufs[p], esems[p]).wait()

    def issue_scatter(ci, p):
        pltpu.async_copy(xbufs[p], shx.at[dst_v.at[ci]], ssems[p], add=True)

        @pl.when(c == 0)
        def _():
            pltpu.async_copy(ebufs[p], shm.at[dst_v.at[ci]], ssems[p],
                             add=True)

        @pl.when(c == 1)
        def _():
            pltpu.async_copy(onesbuf, shm.at[dst_v.at[ci]], ssems[p],
                             add=True)

    def drain_scatter(p):
        pltpu.make_async_copy(xbufs[p], shx.at[dst_v.at[0]], ssems[p]).wait()

        @pl.when(c == 0)
        def _():
            pltpu.make_async_copy(
                ebufs[p], shm.at[dst_v.at[0]], ssems[p]).wait()

        @pl.when(c == 1)
        def _():
            pltpu.make_async_copy(
                onesbuf, shm.at[dst_v.at[0]], ssems[p]).wait()

    def process(ci, p, drain):
        drain_gather(p)
        issue_scatter(ci, p)
        pn = (p + 2) % NBUF
        if drain:
            drain_scatter(pn)
        issue_gather(jnp.minimum(ci + 2, NCH - 1), pn)

    issue_gather(0, 0)
    issue_gather(1, 1)
    process(0, 0, drain=False)
    process(1, 1, drain=True)
    process(2, 2, drain=True)

    def tri_body(j, carry):
        a = 3 * j
        process(a, 0, drain=True)
        process(a + 1, 1, drain=True)
        process(a + 2, 2, drain=True)
        return carry

    lax.fori_loop(1, NCH // 3, tri_body, 0)
    process(NCH - 1, 0, drain=True)

    drain_gather(1)
    drain_gather(2)
    drain_scatter(0)

    plsc.subcore_barrier()

    pltpu.sync_copy(shx.at[pl.ds(r0, ROWS_PER_TILE)],
                    accx_hbm.at[c].at[pl.ds(r0, ROWS_PER_TILE)])
    @pl.when(c == 0)
    def _():
        pltpu.sync_copy(shm.at[pl.ds(r0, ROWS_PER_TILE)],
                        acce_hbm.at[pl.ds(r0, ROWS_PER_TILE)])

    @pl.when(c == 1)
    def _():
        pltpu.sync_copy(shm.at[pl.ds(r0, ROWS_PER_TILE)],
                        accc_hbm.at[pl.ds(r0, ROWS_PER_TILE)])


def _finish_body(accx_ref, acce_ref, accc_ref, wx_ref, we_ref, b_ref, out_ref):
    acc0 = accx_ref[0, :N_NODES]
    acc1 = accx_ref[1, :N_NODES]
    acce = acce_ref[:N_NODES]
    counts = accc_ref[:N_NODES, 0:1]
    sums = jnp.dot(acc0, wx_ref[:D_HALF], preferred_element_type=jnp.float32)
    sums = sums + jnp.dot(acc1, wx_ref[D_HALF:], preferred_element_type=jnp.float32)
    sums = sums + jnp.dot(acce, we_ref[...], preferred_element_type=jnp.float32)
    sums = sums + counts * b_ref[...]
    out_ref[...] = sums / jnp.maximum(counts, 1.0)


_finish = pl.pallas_call(
    _finish_body,
    out_shape=jax.ShapeDtypeStruct((N_NODES, OUT_DIM), jnp.float32),
)


def kernel(x, edge_index, edge_features, W, b):
    src = edge_index[0].astype(jnp.int32).reshape(NS, NCH, CHUNK)
    dst = edge_index[1].astype(jnp.int32).reshape(NS, NCH, CHUNK)
    xsplit = x.reshape(N_NODES, NC, D_HALF).transpose(1, 0, 2)
    z64 = jnp.zeros((ROWS_PER_TILE, D_HALF), jnp.float32)
    z16 = jnp.zeros((ROWS_PER_TILE, 16), jnp.float32)
    ones = jnp.ones((CHUNK, 16), jnp.float32)
    accx, acce, accc = _sc_accumulate(src, dst, xsplit, edge_features,
                                      z64, z16, ones)
    wx = W[:D_FEAT]
    we = W[D_FEAT:]
    return _finish(accx, acce, accc, wx, we, b.reshape(1, OUT_DIM))
